# Initial kernel scaffold; baseline (speedup 1.0000x reference)
import jax
import jax.numpy as jnp
from jax.experimental import pallas as pl


def kernel(x, weight, fc_table, bias):
    raise NotImplementedError("write your pallas kernel here")


# speedup vs baseline: 1.0351x; 1.0351x over previous
import functools

import jax
import jax.numpy as jnp
import numpy as np
from jax import lax
from jax.experimental import pallas as pl
from jax.experimental.pallas import tpu as pltpu
from jax.experimental.pallas import tpu_sc as plsc

_FIELD_DIMS = [100000] * 26
_B = 16384
_F = len(_FIELD_DIMS)
_TOTAL = sum(_FIELD_DIMS)

_NC = 2
_NS = 16
_NW = _NC * _NS
_BPW = _B // _NW
_EPW = _BPW * _F
_LANES = 16

_PERIOD = 208
_OFFSETS = np.concatenate(([0], np.cumsum(_FIELD_DIMS)[:-1])).astype(np.int32)
_OFFS_TILED = np.tile(_OFFSETS, _PERIOD // _F).astype(np.int32)

_CHUNK = 128
_NCHUNK = _EPW // _CHUNK
_INFLIGHT = 8
_NGROUP = _NCHUNK // _INFLIGHT


def _body(x_hbm, w_hbm, offs_hbm, table_hbm, out_hbm,
          xv, wv, offv, idxv, embv, outv, sem):
    wid = lax.axis_index("s") * _NC + lax.axis_index("c")
    ebase = wid * _EPW
    bbase = wid * _BPW

    cp_x = pltpu.async_copy(x_hbm.at[pl.ds(ebase, _EPW)], xv, sem)
    cp_w = pltpu.async_copy(w_hbm.at[pl.ds(ebase, _EPW)], wv, sem)
    cp_o = pltpu.async_copy(offs_hbm, offv, sem)
    cp_x.wait()
    cp_w.wait()
    cp_o.wait()

    def idx_group(g, _):
        base = g * _PERIOD
        for j in range(_PERIOD // _LANES):
            o = base + j * _LANES
            idxv[pl.ds(o, _LANES)] = (
                xv[pl.ds(o, _LANES)] + offv[pl.ds(j * _LANES, _LANES)]
            )
        return 0
    lax.fori_loop(0, _EPW // _PERIOD, idx_group, 0)

    def gather_group(g, _):
        cbase = g * _INFLIGHT
        handles = []
        for j in range(_INFLIGHT):
            off = (cbase + j) * _CHUNK
            handles.append(pltpu.async_copy(
                table_hbm.at[idxv.at[pl.ds(off, _CHUNK)]],
                embv.at[pl.ds(off, _CHUNK)], sem))
        for h in handles:
            h.wait()
        return 0
    lax.fori_loop(0, _NGROUP, gather_group, 0)

    lanes = lax.iota(jnp.int32, _LANES)
    def reduce_group(g, _):
        rowbase = (g * _LANES + lanes) * _F
        acc = jnp.zeros((_LANES,), jnp.float32)
        for f in range(_F):
            e = plsc.load_gather(embv, [rowbase + f])
            w = plsc.load_gather(wv, [rowbase + f])
            acc = acc + e * w
        outv[pl.ds(g * _LANES, _LANES)] = acc
        return 0
    lax.fori_loop(0, _BPW // _LANES, reduce_group, 0)

    pltpu.sync_copy(outv, out_hbm.at[pl.ds(bbase, _BPW)])


@jax.jit
def _sc_lookup(x_flat, w_flat, offs, table):
    mesh = plsc.VectorSubcoreMesh(core_axis_name="c", subcore_axis_name="s")
    f = pl.kernel(
        _body,
        out_type=jax.ShapeDtypeStruct((_B,), jnp.float32),
        mesh=mesh,
        scratch_types=[
            pltpu.VMEM((_EPW,), jnp.int32),
            pltpu.VMEM((_EPW,), jnp.float32),
            pltpu.VMEM((_PERIOD,), jnp.int32),
            pltpu.VMEM((_EPW,), jnp.int32),
            pltpu.VMEM((_EPW,), jnp.float32),
            pltpu.VMEM((_BPW,), jnp.float32),
            pltpu.SemaphoreType.DMA,
        ],
        compiler_params=pltpu.CompilerParams(needs_layout_passes=False),
    )
    return f(x_flat, w_flat, offs, table)


def kernel(x, weight, fc_table, bias):
    x_flat = x.reshape(-1)
    w_flat = weight.reshape(-1)
    table = fc_table.reshape(-1)
    offs = jnp.asarray(_OFFS_TILED)
    out = _sc_lookup(x_flat, w_flat, offs, table)
    return out[:, None] + bias[None, :]


# speedup vs baseline: 3.2730x; 3.1621x over previous
import jax
import jax.numpy as jnp
from jax import lax
from jax.experimental import pallas as pl
from jax.experimental.pallas import tpu as pltpu
from jax.experimental.pallas import tpu_sc as plsc

_FIELD_DIM = 100000
_B = 16384
_F = 26
_TOTAL = _FIELD_DIM * _F

_NC = 2
_NS = 16
_NW = _NC * _NS
_BPW = _B // _NW
_EPW = _BPW * _F
_LANES = 16

_CHUNK = 128
_NCHUNK = _EPW // _CHUNK
_INFLIGHT = 8
_NGROUP = _NCHUNK // _INFLIGHT

_QW = 81152
_TAIL_OFF = _NW * _QW
_TAIL = _TOTAL - _TAIL_OFF


def _relayout_body(tab2_hbm, flat_hbm, buf, tbuf, sem, tsem):
    wid = lax.axis_index("s") * _NC + lax.axis_index("c")
    base = wid * _QW
    cp = pltpu.async_copy(tab2_hbm.at[0, pl.ds(base, _QW)],
                          buf, sem)

    @pl.when(wid == 0)
    def _():
        pltpu.async_copy(tab2_hbm.at[0, pl.ds(_TAIL_OFF, _TAIL)],
                         tbuf, tsem).wait()
        pltpu.async_copy(tbuf, flat_hbm.at[pl.ds(_TAIL_OFF, _TAIL)],
                         tsem).wait()

    cp.wait()
    pltpu.sync_copy(buf, flat_hbm.at[pl.ds(base, _QW)])


def _body(x_hbm, w_hbm, table_hbm, out_hbm,
          xv, wv, idxv, embv, outv, sem):
    wid = lax.axis_index("s") * _NC + lax.axis_index("c")
    bbase = wid * _BPW

    cps = []
    for f in range(_F):
        cps.append(pltpu.async_copy(
            x_hbm.at[pl.ds(f * _B + bbase, _BPW)],
            xv.at[pl.ds(f * _BPW, _BPW)], sem))
        cps.append(pltpu.async_copy(
            w_hbm.at[pl.ds(f * _B + bbase, _BPW)],
            wv.at[pl.ds(f * _BPW, _BPW)], sem))
    for cp in cps:
        cp.wait()

    def idx_group(j, _):
        o = j * _LANES
        for f in range(_F):
            off = jnp.int32(f * _FIELD_DIM)
            idxv[pl.ds(f * _BPW + o, _LANES)] = (
                xv[pl.ds(f * _BPW + o, _LANES)] + off
            )
        return 0
    lax.fori_loop(0, _BPW // _LANES, idx_group, 0)

    def gather_group(g, _):
        cbase = g * _INFLIGHT
        handles = []
        for j in range(_INFLIGHT):
            off = (cbase + j) * _CHUNK
            handles.append(pltpu.async_copy(
                table_hbm.at[idxv.at[pl.ds(off, _CHUNK)]],
                embv.at[pl.ds(off, _CHUNK)], sem))
        for h in handles:
            h.wait()
        return 0
    lax.fori_loop(0, _NGROUP, gather_group, 0)

    def reduce_group(g, _):
        base = g * _LANES
        acc = jnp.zeros((_LANES,), jnp.float32)
        for f in range(_F):
            o = f * _BPW + base
            acc = acc + embv[pl.ds(o, _LANES)] * wv[pl.ds(o, _LANES)]
        outv[pl.ds(base, _LANES)] = acc
        return 0
    lax.fori_loop(0, _BPW // _LANES, reduce_group, 0)

    pltpu.sync_copy(outv, out_hbm.at[pl.ds(bbase, _BPW)])


@jax.jit
def _sc_relayout(table2d):
    mesh = plsc.VectorSubcoreMesh(core_axis_name="c", subcore_axis_name="s")
    f = pl.kernel(
        _relayout_body,
        out_type=jax.ShapeDtypeStruct((_TOTAL,), jnp.float32),
        mesh=mesh,
        scratch_types=[
            pltpu.VMEM((_QW,), jnp.float32),
            pltpu.VMEM((_TAIL,), jnp.float32),
            pltpu.SemaphoreType.DMA,
            pltpu.SemaphoreType.DMA,
        ],
    )
    return f(table2d)


@jax.jit
def _sc_lookup(x_t, w_t, table):
    mesh = plsc.VectorSubcoreMesh(core_axis_name="c", subcore_axis_name="s")
    f = pl.kernel(
        _body,
        out_type=jax.ShapeDtypeStruct((_B,), jnp.float32),
        mesh=mesh,
        scratch_types=[
            pltpu.VMEM((_EPW,), jnp.int32),
            pltpu.VMEM((_EPW,), jnp.float32),
            pltpu.VMEM((_EPW,), jnp.int32),
            pltpu.VMEM((_EPW,), jnp.float32),
            pltpu.VMEM((_BPW,), jnp.float32),
            pltpu.SemaphoreType.DMA,
        ],
        compiler_params=pltpu.CompilerParams(needs_layout_passes=False),
    )
    return f(x_t, w_t, table)


def kernel(x, weight, fc_table, bias):
    x_t = x.T.reshape(-1)
    w_t = jnp.transpose(weight, (1, 2, 0)).reshape(-1)
    table = _sc_relayout(fc_table.T)
    out = _sc_lookup(x_t, w_t, table)
    return out[:, None] + bias[None, :]
